# baseline (speedup 1.0000x reference)
import functools

import numpy as np
import jax
import jax.numpy as jnp
from jax import lax
from jax.experimental import pallas as pl
from jax.experimental.pallas import tpu as pltpu
import jax.experimental.pallas.tpu_sc as plsc

_B, _L, _D = 4096, 50, 64
_NC, _NS = 2, 16
_NW = _NC * _NS
_BPW = _B // _NW
_PPW = _BPW * _L
_CHUNK = 128
_NCHUNK = _PPW // _CHUNK
_NBUF = 4


def _body(idx0, idx1, dst, zeros, tab0, tab1, out,
          idxv, dstv, rows, acc0, acc1, gsem, ssem, zsem):
    sid = lax.axis_index("s")
    wid = sid * _NC + lax.axis_index("c")
    base = wid * _BPW
    accs = (acc0, acc1)
    tables = (tab0, tab1)

    for f, idx_hbm in enumerate((idx0, idx1)):
        pltpu.sync_copy(idx_hbm.at[wid], idxv.at[f])
    pltpu.sync_copy(dst.at[sid], dstv)
    my = pl.ds(sid * _BPW, _BPW)
    for f in range(2):
        pltpu.async_copy(zeros, accs[f].at[my], zsem)
    for f in range(2):
        pltpu.make_async_copy(zeros, accs[f].at[my], zsem).wait()

    pending = [False] * _NBUF

    def gather(f, j, slot):
        pltpu.async_copy(tables[f].at[idxv.at[f, j]], rows.at[slot],
                         gsem.at[slot])

    def wait_gather(f, slot):
        pltpu.make_async_copy(tables[f].at[idxv.at[f, 0]], rows.at[slot],
                              gsem.at[slot]).wait()

    def scatter(f, j, slot):
        pltpu.async_copy(rows.at[slot], accs[f].at[dstv.at[j]],
                         ssem.at[slot], add=True)

    def wait_scatter(f, slot):
        pltpu.make_async_copy(rows.at[slot], accs[f].at[dstv.at[0]],
                              ssem.at[slot]).wait()

    for f in range(2):
        for b in range(_NBUF):
            if pending[b]:
                wait_scatter(f - 1, b)
                pending[b] = False
            gather(f, b, b)
        wait_gather(f, 0)
        scatter(f, 0, 0)

        def step(j, carry):
            p = j % _NBUF
            p1 = (j - 1) % _NBUF
            wait_gather(f, p)
            scatter(f, j, p)
            wait_scatter(f, p1)
            gather(f, j - 1 + _NBUF, p1)
            return carry

        lax.fori_loop(1, _NCHUNK - _NBUF + 1, step, 0, unroll=2)

        for j in range(_NCHUNK - _NBUF + 1, _NCHUNK):
            p = j % _NBUF
            wait_gather(f, p)
            scatter(f, j, p)
        for j in range(_NCHUNK - _NBUF, _NCHUNK):
            pending[j % _NBUF] = True

    for b in range(_NBUF):
        if pending[b]:
            wait_scatter(1, b)
            pending[b] = False
    for f in range(2):
        pltpu.sync_copy(accs[f].at[my],
                        out.at[pl.ds(base, _BPW), pl.ds(f * _D, _D)])


_LOCAL = (np.arange(_PPW, dtype=np.int32) // _L).reshape(_NCHUNK, _CHUNK)
_DST = (np.arange(_NS, dtype=np.int32)[:, None, None] * _BPW
        + _LOCAL[None]).astype(np.int32)
_ZEROS = np.zeros((_BPW, _D), np.float32)

_NREP = 8
_REP_OFF = ((np.arange(_NCHUNK, dtype=np.int32) % _NREP) * 1000)[None, :, None]


def _make_pair_kernel():
    acc_t = pltpu.VMEM_SHARED((_NS * _BPW, _D), jnp.float32)
    return pl.kernel(
        _body,
        out_type=jax.ShapeDtypeStruct((_B, 2 * _D), jnp.float32),
        mesh=plsc.VectorSubcoreMesh(core_axis_name="c", subcore_axis_name="s"),
        compiler_params=pltpu.CompilerParams(use_tc_tiling_on_sc=False),
        scratch_types=[
            pltpu.VMEM((2, _NCHUNK, _CHUNK), jnp.int32),
            pltpu.VMEM((_NCHUNK, _CHUNK), jnp.int32),
            pltpu.VMEM((_NBUF, _CHUNK, _D), jnp.float32),
            acc_t, acc_t,
            pltpu.SemaphoreType.DMA((_NBUF,)),
            pltpu.SemaphoreType.DMA((_NBUF,)),
            pltpu.SemaphoreType.DMA,
        ],
    )


@jax.jit
def kernel(vgids, vsids, vcids, vgprices,
           goods_table, shop_table, cate_table, price_table):
    shape3 = (_NW, _NCHUNK, _CHUNK)
    gidx = vgids.astype(jnp.int32).reshape(shape3)
    sidx = vsids.astype(jnp.int32).reshape(shape3)
    cidx = vcids.astype(jnp.int32).reshape(shape3)
    pidx = vgprices.astype(jnp.int32).reshape(shape3)
    dst = jnp.asarray(_DST)
    zeros = jnp.asarray(_ZEROS)

    rep_off = jnp.asarray(_REP_OFF)
    cidx = cidx + rep_off
    pidx = pidx + rep_off
    cate8 = jnp.tile(cate_table, (_NREP, 1))
    price8 = jnp.tile(price_table, (_NREP, 1))

    run = _make_pair_kernel()
    out_cp = run(cidx, pidx, dst, zeros, cate8, price8)
    out_gs = run(gidx, sidx, dst, zeros, goods_table, shop_table)
    return jnp.concatenate([out_gs, out_cp], axis=1)
